# baseline (speedup 1.0000x reference)
import functools
import math

import jax
import jax.numpy as jnp
import numpy as np
from jax import lax
from jax.experimental import pallas as pl
from jax.experimental.pallas import tpu as pltpu
from jax.experimental.pallas import tpu_sc as plsc


def _pe_flat(S, D):
    pe = np.zeros((S, D), dtype=np.float32)
    position = np.arange(0, S, dtype=np.float32)[:, None]
    div_term = np.exp(
        np.arange(0, D, 2, dtype=np.float32) * -(math.log(10000.0) / D))
    pe[:, 0::2] = np.sin(position * div_term)
    pe[:, 1::2] = np.cos(position * div_term)
    return pe.reshape(-1)


_TILED = pltpu.CompilerParams(
    use_tc_tiling_on_sc=True, needs_layout_passes=False)
_L = 16


def _worker_id():
    info = plsc.get_sparse_core_info()
    return lax.axis_index("s") * info.num_cores + lax.axis_index("c")


def _mesh():
    return plsc.VectorSubcoreMesh(core_axis_name="c", subcore_axis_name="s")


@functools.lru_cache(maxsize=None)
def _make_k1(V, D):
    NW = 32
    NB = (V + 127) // 128
    HALF = PER = (NB + 2 * NW - 1) // (2 * NW)
    R2R = NB * (128 // 2)
    SCALE = float(np.sqrt(D))

    mesh = _mesh()

    @functools.partial(
        pl.kernel, mesh=mesh, compiler_params=_TILED,
        out_type=jax.ShapeDtypeStruct((R2R, 2 * D), jnp.float32),
        scratch_types=[
            pltpu.VMEM((D, 128), jnp.float32),
            pltpu.VMEM((D, 128), jnp.float32),
            pltpu.VMEM((D, 128), jnp.float32),
            pltpu.VMEM((D, 128), jnp.float32),
            pltpu.SemaphoreType.DMA,
            pltpu.SemaphoreType.DMA,
            pltpu.SemaphoreType.DMA,
            pltpu.SemaphoreType.DMA,
        ],
    )
    def k1(tin, r2, in0, in1, o0, o1, g0, g1, w0, w1):
        wid = _worker_id()
        ins, outs = (in0, in1), (o0, o1)
        gsems, osems = (g0, g1), (w0, w1)
        rowv = [lax.iota(jnp.int32, _L) + 16 * m for m in range(4)]

        for p in range(2):
            pltpu.async_copy(
                tin.at[:, pl.ds((wid + NW * p) * 128, 128)], ins[p], gsems[p])

        def halfstep(i, p):
            k2i = 2 * i + p
            blk = wid + NW * k2i

            @pl.when(blk < NB)
            def _():
                inb, outb = ins[p], outs[p]
                pltpu.make_async_copy(
                    tin.at[:, pl.ds(0, 128)], inb, gsems[p]).wait()

                @pl.when(k2i >= 2)
                def _():
                    pltpu.make_async_copy(
                        outb, r2.at[pl.ds(0, D)], osems[p]).wait()

                def rows(r, carry):
                    for half in range(2):
                        csp = jnp.full((_L,), 2 * r + half, jnp.int32)
                        for m in range(4):
                            g = plsc.load_gather(inb, [rowv[m], csp])
                            outb[r, pl.ds((half * 4 + m) * _L, _L)] = g * SCALE
                    return carry

                lax.fori_loop(0, D, rows, 0, unroll=False)
                pltpu.async_copy(outb, r2.at[pl.ds(blk * D, D)], osems[p])

                @pl.when(blk + 2 * NW < NB)
                def _():
                    pltpu.async_copy(
                        tin.at[:, pl.ds((blk + 2 * NW) * 128, 128)],
                        inb, gsems[p])

        def step(i, carry):
            halfstep(i, 0)
            halfstep(i, 1)
            return carry

        lax.fori_loop(0, HALF, step, 0, unroll=False)
        pltpu.make_async_copy(o0, r2.at[pl.ds(0, D)], w0).wait()
        pltpu.make_async_copy(o1, r2.at[pl.ds(0, D)], w1).wait()

    return k1


@functools.lru_cache(maxsize=None)
def _make_k2(S, B, D, R2R):
    NW = 32
    CB = B // NW
    mesh = _mesh()

    @functools.partial(
        pl.kernel, mesh=mesh, compiler_params=_TILED,
        out_type=jax.ShapeDtypeStruct((S, D, B), jnp.float32),
        scratch_types=[
            pltpu.VMEM((S, CB), jnp.int32),
            pltpu.VMEM((S * D,), jnp.float32),
            pltpu.VMEM((2, CB), jnp.int32),
            pltpu.VMEM((2, CB), jnp.int32),
            pltpu.VMEM((CB, 2 * D), jnp.float32),
            pltpu.VMEM((CB, 2 * D), jnp.float32),
            pltpu.VMEM((D, CB), jnp.float32),
            pltpu.VMEM((D, CB), jnp.float32),
            pltpu.SemaphoreType.DMA,
            pltpu.SemaphoreType.DMA,
            pltpu.SemaphoreType.DMA,
            pltpu.SemaphoreType.DMA,
        ],
    )
    def k2(src, r2, pe, out, idxv, pev, i2b, hob, pb0, pb1, tb0, tb1,
           g0, g1, w0, w1):
        wid = _worker_id()
        col0 = wid * CB
        pairs, tbufs = (pb0, pb1), (tb0, tb1)
        gsems, osems = (g0, g1), (w0, w1)
        NQ = CB // _L
        rowv = [lax.iota(jnp.int32, _L) + 16 * q for q in range(NQ)]

        pltpu.sync_copy(src.at[:, pl.ds(col0, CB)], idxv)
        pltpu.sync_copy(pe, pev)

        def prep_and_fire(s, p):
            for q in range(NQ):
                v = idxv[s, pl.ds(16 * q, _L)]
                i2b[p, pl.ds(16 * q, _L)] = lax.shift_right_logical(v, 1)
                hob[p, pl.ds(16 * q, _L)] = (v & 1) * D
            pltpu.async_copy(r2.at[i2b.at[p]], pairs[p], gsems[p])

        prep_and_fire(0, 0)
        prep_and_fire(1, 1)

        def halfstep(i, p):
            s = 2 * i + p
            pb, tb = pairs[p], tbufs[p]
            pltpu.make_async_copy(
                r2.at[pl.ds(0, CB)], pb, gsems[p]).wait()

            @pl.when(s >= 2)
            def _():
                pltpu.make_async_copy(
                    tb, out.at[0, :, pl.ds(col0, CB)], osems[p]).wait()

            hvec = [hob[p, pl.ds(16 * q, _L)] for q in range(NQ)]

            def jloop(j, carry):
                pej = plsc.load_gather(
                    pev, [jnp.full((_L,), s * D + j, jnp.int32)])
                jsp = jnp.full((_L,), j, jnp.int32)
                for q in range(NQ):
                    val = plsc.load_gather(pb, [rowv[q], hvec[q] + jsp])
                    tb[j, pl.ds(16 * q, _L)] = val + pej
                return carry

            lax.fori_loop(0, D, jloop, 0, unroll=False)
            pltpu.async_copy(tb, out.at[s, :, pl.ds(col0, CB)], osems[p])

            @pl.when(s + 2 < S)
            def _():
                prep_and_fire(s + 2, p)

        def step(i, carry):
            halfstep(i, 0)
            halfstep(i, 1)
            return carry

        lax.fori_loop(0, S // 2, step, 0, unroll=False)
        pltpu.make_async_copy(tb0, out.at[0, :, pl.ds(col0, CB)], w0).wait()
        pltpu.make_async_copy(tb1, out.at[0, :, pl.ds(col0, CB)], w1).wait()

    return k2


def kernel(src, table, step):
    del step
    S, B = src.shape
    V, D = table.shape
    pe = jnp.asarray(_pe_flat(S, D))
    table_t = jnp.swapaxes(table, 0, 1)
    r2 = _make_k1(V, D)(table_t)
    p_out = _make_k2(S, B, D, r2.shape[0])(src.astype(jnp.int32), r2, pe)
    return jnp.swapaxes(p_out, 1, 2)
